# baseline (speedup 1.0000x reference)
import functools

import jax
import jax.numpy as jnp
from jax import lax
from jax.experimental import pallas as pl
from jax.experimental.pallas import tpu as pltpu
from jax.experimental.pallas import tpu_sc as plsc

K = 20
BN = 8192
F = 128
ROWS = 512
N_TILES = BN // ROWS
EDGES = BN * K
NC, NS = 2, 16
NW = NC * NS
E_PER_W = EDGES // NW
GCHUNK = 128
N_CHUNKS = E_PER_W // GCHUNK
CROWS = 128


NCHUNK = 64
DEPTH = 4


def _dist_topk_body(p_ref, pt_ref, f_ref, wd_ref, wb_ref, b1_ref,
                    knn_ref, a_ref, bm_ref):
    p_blk = p_ref[...]
    pt = pt_ref[...]
    sq_row = jnp.sum(p_blk * p_blk, axis=1, keepdims=True)
    sq_col = jnp.sum(pt * pt, axis=0, keepdims=True) * 0.25
    dot2 = jnp.dot(p_blk, pt, preferred_element_type=jnp.float32)
    d2 = (sq_row + sq_col) + dot2

    big = jnp.float32(65536.0)
    inf = jnp.float32(jnp.inf)
    lane = lax.broadcasted_iota(jnp.int32, (ROWS, 128), 1).astype(jnp.float32)

    chunks = [d2[:, c * 128:(c + 1) * 128] for c in range(NCHUNK)]

    Ms, Is = [], []
    for _ in range(DEPTH):
        m = chunks[0]
        for c in range(1, NCHUNK):
            m = jnp.minimum(m, chunks[c])
        pos = jnp.full((ROWS, 128), big, jnp.float32)
        for c in range(NCHUNK - 1, -1, -1):
            pos = jnp.where(chunks[c] == m, jnp.float32(c), pos)
        chunks = [jnp.where(pos == jnp.float32(c), inf, chunks[c])
                  for c in range(NCHUNK)]
        Ms.append(m)
        Is.append(pos * 128.0 + lane)

    S = chunks[0]
    for c in range(1, NCHUNK):
        S = jnp.minimum(S, chunks[c])

    C, CI = Ms[0], Is[0]
    U = jnp.zeros((ROWS, 128), jnp.float32)
    cols = []
    for _ in range(K):
        m = jnp.min(C, axis=1, keepdims=True)
        cand = jnp.where(C == m, CI, big)
        idx = jnp.min(cand, axis=1, keepdims=True)
        cols.append(idx)
        sel = cand == idx
        U = U + sel.astype(jnp.float32)
        nc = jnp.full((ROWS, 128), inf, jnp.float32)
        ni = jnp.full((ROWS, 128), big, jnp.float32)
        nc = jnp.where(U == jnp.float32(DEPTH), S, nc)
        for l in range(DEPTH - 1, 0, -1):
            at = U == jnp.float32(l)
            nc = jnp.where(at, Ms[l], nc)
            ni = jnp.where(at, Is[l], ni)
        C = jnp.where(sel, nc, C)
        CI = jnp.where(sel, ni, CI)
    knn_ref[...] = jnp.concatenate(cols, axis=1).astype(jnp.int32)
    badidx = cols[0]
    for t in range(1, K):
        badidx = jnp.maximum(badidx, cols[t])

    @pl.when(jnp.max(badidx) >= jnp.float32(BN))
    def _fallback():
        iota = lax.broadcasted_iota(jnp.int32, (ROWS, BN), 1).astype(
            jnp.float32)
        dd = d2
        fcols = []
        for _ in range(K):
            fm = jnp.min(dd, axis=1, keepdims=True)
            fi = jnp.min(jnp.where(dd == fm, iota, big), axis=1,
                         keepdims=True)
            fcols.append(fi)
            dd = jnp.where(iota == fi, inf, dd)
        knn_ref[...] = jnp.concatenate(fcols, axis=1).astype(jnp.int32)

    f_blk = f_ref[...]
    a_ref[...] = jnp.dot(f_blk, wd_ref[...],
                         preferred_element_type=jnp.float32) + b1_ref[...]
    bm_ref[...] = jnp.dot(f_blk, wb_ref[...],
                          preferred_element_type=jnp.float32)


def _edge_mlp_body(g_ref, a_ref, w2t_ref, b2_ref, o_ref):
    a_rep = jnp.broadcast_to(a_ref[...][:, None, :], (CROWS, K, F))
    s = jax.nn.relu(a_rep.reshape(CROWS * K, F) + g_ref[...])
    h = jnp.dot(s, w2t_ref[...], preferred_element_type=jnp.float32)
    o_ref[...] = jnp.max(h.reshape(CROWS, K, F), axis=1) + b2_ref[...]


NBUF = 4


def _sc_gather(bm_hbm, idx_hbm, out_hbm, idx_v, b0, b1, b2, b3, sem):
    wid = lax.axis_index("s") * NC + lax.axis_index("c")
    base = wid * E_PER_W
    bufs = (b0, b1, b2, b3)
    pltpu.sync_copy(idx_hbm.at[pl.ds(base, E_PER_W)], idx_v)

    def step(g, carry):
        hs = []
        for b in range(NBUF):
            off = (g * NBUF + b) * GCHUNK
            hs.append(pltpu.async_copy(
                bm_hbm.at[idx_v.at[pl.ds(off, GCHUNK)]], bufs[b], sem))
        for h in hs:
            h.wait()
        ws = []
        for b in range(NBUF):
            off = (g * NBUF + b) * GCHUNK
            ws.append(pltpu.async_copy(
                bufs[b], out_hbm.at[pl.ds(base + off, GCHUNK)], sem))
        for w in ws:
            w.wait()
        return carry

    lax.fori_loop(0, N_CHUNKS // NBUF, step, 0)


def kernel(pos, features, W1, b1, W2, b2):
    B, N, _ = pos.shape
    p = pos.reshape(BN, 3)
    f = features.reshape(BN, F)
    P = jnp.pad(p, ((0, 0), (0, 5)))
    PT = -2.0 * P.T
    W1a = W1[:, :F]
    W1b = W1[:, F:]
    Wd = (W1a - W1b).T
    Wb = W1b.T
    W2T = W2.T
    b1r = b1.reshape(1, F)
    b2r = b2.reshape(1, F)

    knn, A, Bm = pl.pallas_call(
        _dist_topk_body,
        grid=(N_TILES,),
        in_specs=[
            pl.BlockSpec((ROWS, 8), lambda i: (i, 0)),
            pl.BlockSpec((8, BN), lambda i: (0, 0)),
            pl.BlockSpec((ROWS, F), lambda i: (i, 0)),
            pl.BlockSpec((F, F), lambda i: (0, 0)),
            pl.BlockSpec((F, F), lambda i: (0, 0)),
            pl.BlockSpec((1, F), lambda i: (0, 0)),
        ],
        out_specs=[
            pl.BlockSpec((ROWS, K), lambda i: (i, 0)),
            pl.BlockSpec((ROWS, F), lambda i: (i, 0)),
            pl.BlockSpec((ROWS, F), lambda i: (i, 0)),
        ],
        out_shape=[
            jax.ShapeDtypeStruct((BN, K), jnp.int32),
            jax.ShapeDtypeStruct((BN, F), jnp.float32),
            jax.ShapeDtypeStruct((BN, F), jnp.float32),
        ],
    )(P, PT, f, Wd, Wb, b1r)

    idx_flat = knn.reshape(EDGES)

    gather = functools.partial(
        pl.kernel,
        out_type=jax.ShapeDtypeStruct((EDGES, F), jnp.float32),
        mesh=plsc.VectorSubcoreMesh(core_axis_name="c", subcore_axis_name="s"),
        scratch_types=[
            pltpu.VMEM((E_PER_W,), jnp.int32),
            pltpu.VMEM((GCHUNK, F), jnp.float32),
            pltpu.VMEM((GCHUNK, F), jnp.float32),
            pltpu.VMEM((GCHUNK, F), jnp.float32),
            pltpu.VMEM((GCHUNK, F), jnp.float32),
            pltpu.SemaphoreType.DMA,
        ],
    )(_sc_gather)
    G = gather(Bm, idx_flat)

    out = pl.pallas_call(
        _edge_mlp_body,
        grid=(BN // CROWS,),
        in_specs=[
            pl.BlockSpec((CROWS * K, F), lambda i: (i, 0)),
            pl.BlockSpec((CROWS, F), lambda i: (i, 0)),
            pl.BlockSpec((F, F), lambda i: (0, 0)),
            pl.BlockSpec((1, F), lambda i: (0, 0)),
        ],
        out_specs=pl.BlockSpec((CROWS, F), lambda i: (i, 0)),
        out_shape=jax.ShapeDtypeStruct((BN, F), jnp.float32),
    )(G, A, W2T, b2r)

    return out.reshape(B, N, F)


# speedup vs baseline: 1.0530x; 1.0530x over previous
import functools

import jax
import jax.numpy as jnp
from jax import lax
from jax.experimental import pallas as pl
from jax.experimental.pallas import tpu as pltpu
from jax.experimental.pallas import tpu_sc as plsc

K = 20
BN = 8192
F = 128
ROWS = 256
N_TILES = BN // ROWS
EDGES = BN * K
NC, NS = 2, 16
NW = NC * NS
E_PER_W = EDGES // NW
GCHUNK = 128
N_CHUNKS = E_PER_W // GCHUNK
CROWS = 256


NCHUNK = 64
DEPTH = 4


def _dist_topk_body(p_ref, pt_ref, f_ref, wd_ref, wb_ref, b1_ref,
                    knn_ref, a_ref, bm_ref):
    p_blk = p_ref[...]
    pt = pt_ref[...]
    sq_row = jnp.sum(p_blk * p_blk, axis=1, keepdims=True)
    sq_col = jnp.sum(pt * pt, axis=0, keepdims=True) * 0.25
    dot2 = jnp.dot(p_blk, pt, preferred_element_type=jnp.float32)
    d2 = (sq_row + sq_col) + dot2

    big = jnp.float32(65536.0)
    inf = jnp.float32(jnp.inf)
    lane = lax.broadcasted_iota(jnp.int32, (ROWS, 128), 1).astype(jnp.float32)

    chunks = [d2[:, c * 128:(c + 1) * 128] for c in range(NCHUNK)]

    Ms, Is = [], []
    for _ in range(DEPTH):
        m = chunks[0]
        for c in range(1, NCHUNK):
            m = jnp.minimum(m, chunks[c])
        pos = jnp.full((ROWS, 128), big, jnp.float32)
        for c in range(NCHUNK - 1, -1, -1):
            pos = jnp.where(chunks[c] == m, jnp.float32(c), pos)
        chunks = [jnp.where(pos == jnp.float32(c), inf, chunks[c])
                  for c in range(NCHUNK)]
        Ms.append(m)
        Is.append(pos * 128.0 + lane)

    S = chunks[0]
    for c in range(1, NCHUNK):
        S = jnp.minimum(S, chunks[c])

    C, CI = Ms[0], Is[0]
    U = jnp.zeros((ROWS, 128), jnp.float32)
    cols = []
    for _ in range(K):
        m = jnp.min(C, axis=1, keepdims=True)
        cand = jnp.where(C == m, CI, big)
        idx = jnp.min(cand, axis=1, keepdims=True)
        cols.append(idx)
        sel = cand == idx
        U = U + sel.astype(jnp.float32)
        nc = jnp.full((ROWS, 128), inf, jnp.float32)
        ni = jnp.full((ROWS, 128), big, jnp.float32)
        nc = jnp.where(U == jnp.float32(DEPTH), S, nc)
        for l in range(DEPTH - 1, 0, -1):
            at = U == jnp.float32(l)
            nc = jnp.where(at, Ms[l], nc)
            ni = jnp.where(at, Is[l], ni)
        C = jnp.where(sel, nc, C)
        CI = jnp.where(sel, ni, CI)
    knn_ref[...] = jnp.concatenate(cols, axis=1).astype(jnp.int32)
    badidx = cols[0]
    for t in range(1, K):
        badidx = jnp.maximum(badidx, cols[t])

    @pl.when(jnp.max(badidx) >= jnp.float32(BN))
    def _fallback():
        iota = lax.broadcasted_iota(jnp.int32, (ROWS, BN), 1).astype(
            jnp.float32)
        dd = d2
        fcols = []
        for _ in range(K):
            fm = jnp.min(dd, axis=1, keepdims=True)
            fi = jnp.min(jnp.where(dd == fm, iota, big), axis=1,
                         keepdims=True)
            fcols.append(fi)
            dd = jnp.where(iota == fi, inf, dd)
        knn_ref[...] = jnp.concatenate(fcols, axis=1).astype(jnp.int32)

    f_blk = f_ref[...]
    a_ref[...] = jnp.dot(f_blk, wd_ref[...],
                         preferred_element_type=jnp.float32) + b1_ref[...]
    bm_ref[...] = jnp.dot(f_blk, wb_ref[...],
                          preferred_element_type=jnp.float32)


def _edge_mlp_body(g_ref, a_ref, w2t_ref, b2_ref, o_ref):
    a_rep = jnp.broadcast_to(a_ref[...][:, None, :], (CROWS, K, F))
    s = jax.nn.relu(a_rep.reshape(CROWS * K, F) + g_ref[...])
    h = jnp.dot(s, w2t_ref[...], preferred_element_type=jnp.float32)
    o_ref[...] = jnp.max(h.reshape(CROWS, K, F), axis=1) + b2_ref[...]


NBUF = 4


def _sc_gather(bm_hbm, idx_hbm, out_hbm, idx_v, b0, b1, b2, b3, sem):
    wid = lax.axis_index("s") * NC + lax.axis_index("c")
    base = wid * E_PER_W
    bufs = (b0, b1, b2, b3)
    pltpu.sync_copy(idx_hbm.at[pl.ds(base, E_PER_W)], idx_v)

    def step(g, carry):
        hs = []
        for b in range(NBUF):
            off = (g * NBUF + b) * GCHUNK
            hs.append(pltpu.async_copy(
                bm_hbm.at[idx_v.at[pl.ds(off, GCHUNK)]], bufs[b], sem))
        for h in hs:
            h.wait()
        ws = []
        for b in range(NBUF):
            off = (g * NBUF + b) * GCHUNK
            ws.append(pltpu.async_copy(
                bufs[b], out_hbm.at[pl.ds(base + off, GCHUNK)], sem))
        for w in ws:
            w.wait()
        return carry

    lax.fori_loop(0, N_CHUNKS // NBUF, step, 0)


def kernel(pos, features, W1, b1, W2, b2):
    B, N, _ = pos.shape
    p = pos.reshape(BN, 3)
    f = features.reshape(BN, F)
    P = jnp.pad(p, ((0, 0), (0, 5)))
    PT = -2.0 * P.T
    W1a = W1[:, :F]
    W1b = W1[:, F:]
    Wd = (W1a - W1b).T
    Wb = W1b.T
    W2T = W2.T
    b1r = b1.reshape(1, F)
    b2r = b2.reshape(1, F)

    knn, A, Bm = pl.pallas_call(
        _dist_topk_body,
        grid=(N_TILES,),
        in_specs=[
            pl.BlockSpec((ROWS, 8), lambda i: (i, 0)),
            pl.BlockSpec((8, BN), lambda i: (0, 0)),
            pl.BlockSpec((ROWS, F), lambda i: (i, 0)),
            pl.BlockSpec((F, F), lambda i: (0, 0)),
            pl.BlockSpec((F, F), lambda i: (0, 0)),
            pl.BlockSpec((1, F), lambda i: (0, 0)),
        ],
        out_specs=[
            pl.BlockSpec((ROWS, K), lambda i: (i, 0)),
            pl.BlockSpec((ROWS, F), lambda i: (i, 0)),
            pl.BlockSpec((ROWS, F), lambda i: (i, 0)),
        ],
        out_shape=[
            jax.ShapeDtypeStruct((BN, K), jnp.int32),
            jax.ShapeDtypeStruct((BN, F), jnp.float32),
            jax.ShapeDtypeStruct((BN, F), jnp.float32),
        ],
    )(P, PT, f, Wd, Wb, b1r)

    idx_flat = knn.reshape(EDGES)

    gather = functools.partial(
        pl.kernel,
        out_type=jax.ShapeDtypeStruct((EDGES, F), jnp.float32),
        mesh=plsc.VectorSubcoreMesh(core_axis_name="c", subcore_axis_name="s"),
        scratch_types=[
            pltpu.VMEM((E_PER_W,), jnp.int32),
            pltpu.VMEM((GCHUNK, F), jnp.float32),
            pltpu.VMEM((GCHUNK, F), jnp.float32),
            pltpu.VMEM((GCHUNK, F), jnp.float32),
            pltpu.VMEM((GCHUNK, F), jnp.float32),
            pltpu.SemaphoreType.DMA,
        ],
    )(_sc_gather)
    G = gather(Bm, idx_flat)

    out = pl.pallas_call(
        _edge_mlp_body,
        grid=(BN // CROWS,),
        in_specs=[
            pl.BlockSpec((CROWS * K, F), lambda i: (i, 0)),
            pl.BlockSpec((CROWS, F), lambda i: (i, 0)),
            pl.BlockSpec((F, F), lambda i: (0, 0)),
            pl.BlockSpec((1, F), lambda i: (0, 0)),
        ],
        out_specs=pl.BlockSpec((CROWS, F), lambda i: (i, 0)),
        out_shape=jax.ShapeDtypeStruct((BN, F), jnp.float32),
    )(G, A, W2T, b2r)

    return out.reshape(B, N, F)


# speedup vs baseline: 1.0642x; 1.0106x over previous
import functools

import jax
import jax.numpy as jnp
from jax import lax
from jax.experimental import pallas as pl
from jax.experimental.pallas import tpu as pltpu
from jax.experimental.pallas import tpu_sc as plsc

K = 20
BN = 8192
F = 128
ROWS = 256
N_TILES = BN // ROWS
EDGES = BN * K
NC, NS = 2, 16
NW = NC * NS
E_PER_W = EDGES // NW
GCHUNK = 128
N_CHUNKS = E_PER_W // GCHUNK
CROWS = 512


NCHUNK = 64
DEPTH = 4


def _dist_topk_body(p_ref, pt_ref, f_ref, wd_ref, wb_ref, b1_ref,
                    knn_ref, a_ref, bm_ref):
    p_blk = p_ref[...]
    pt = pt_ref[...]
    sq_row = jnp.sum(p_blk * p_blk, axis=1, keepdims=True)
    sq_col = jnp.sum(pt * pt, axis=0, keepdims=True) * 0.25
    dot2 = jnp.dot(p_blk, pt, preferred_element_type=jnp.float32)
    d2 = (sq_row + sq_col) + dot2

    big = jnp.float32(65536.0)
    inf = jnp.float32(jnp.inf)
    lane = lax.broadcasted_iota(jnp.int32, (ROWS, 128), 1).astype(jnp.float32)

    chunks = [d2[:, c * 128:(c + 1) * 128] for c in range(NCHUNK)]

    Ms, Is = [], []
    for _ in range(DEPTH):
        m = chunks[0]
        for c in range(1, NCHUNK):
            m = jnp.minimum(m, chunks[c])
        pos = jnp.full((ROWS, 128), big, jnp.float32)
        for c in range(NCHUNK - 1, -1, -1):
            pos = jnp.where(chunks[c] == m, jnp.float32(c), pos)
        chunks = [jnp.where(pos == jnp.float32(c), inf, chunks[c])
                  for c in range(NCHUNK)]
        Ms.append(m)
        Is.append(pos * 128.0 + lane)

    S = chunks[0]
    for c in range(1, NCHUNK):
        S = jnp.minimum(S, chunks[c])

    C, CI = Ms[0], Is[0]
    U = jnp.zeros((ROWS, 128), jnp.float32)
    cols = []
    for _ in range(K):
        m = jnp.min(C, axis=1, keepdims=True)
        cand = jnp.where(C == m, CI, big)
        idx = jnp.min(cand, axis=1, keepdims=True)
        cols.append(idx)
        sel = cand == idx
        U = U + sel.astype(jnp.float32)
        nc = jnp.full((ROWS, 128), inf, jnp.float32)
        ni = jnp.full((ROWS, 128), big, jnp.float32)
        nc = jnp.where(U == jnp.float32(DEPTH), S, nc)
        for l in range(DEPTH - 1, 0, -1):
            at = U == jnp.float32(l)
            nc = jnp.where(at, Ms[l], nc)
            ni = jnp.where(at, Is[l], ni)
        C = jnp.where(sel, nc, C)
        CI = jnp.where(sel, ni, CI)
    knn_ref[...] = jnp.concatenate(cols, axis=1).astype(jnp.int32)
    badidx = cols[0]
    for t in range(1, K):
        badidx = jnp.maximum(badidx, cols[t])

    @pl.when(jnp.max(badidx) >= jnp.float32(BN))
    def _fallback():
        iota = lax.broadcasted_iota(jnp.int32, (ROWS, BN), 1).astype(
            jnp.float32)
        dd = d2
        fcols = []
        for _ in range(K):
            fm = jnp.min(dd, axis=1, keepdims=True)
            fi = jnp.min(jnp.where(dd == fm, iota, big), axis=1,
                         keepdims=True)
            fcols.append(fi)
            dd = jnp.where(iota == fi, inf, dd)
        knn_ref[...] = jnp.concatenate(fcols, axis=1).astype(jnp.int32)

    f_blk = f_ref[...]
    a_ref[...] = jnp.dot(f_blk, wd_ref[...],
                         preferred_element_type=jnp.float32) + b1_ref[...]
    bm_ref[...] = jnp.dot(f_blk, wb_ref[...],
                          preferred_element_type=jnp.float32)


def _edge_mlp_body(g_ref, a_ref, w2t_ref, b2_ref, o_ref):
    a_rep = jnp.broadcast_to(a_ref[...][:, None, :], (CROWS, K, F))
    s = jax.nn.relu(a_rep.reshape(CROWS * K, F) + g_ref[...])
    h = jnp.dot(s, w2t_ref[...], preferred_element_type=jnp.float32)
    o_ref[...] = jnp.max(h.reshape(CROWS, K, F), axis=1) + b2_ref[...]


NBUF = 4


def _sc_gather(bm_hbm, idx_hbm, out_hbm, idx_v, b0, b1, b2, b3, sem):
    wid = lax.axis_index("s") * NC + lax.axis_index("c")
    base = wid * E_PER_W
    bufs = (b0, b1, b2, b3)
    pltpu.sync_copy(idx_hbm.at[pl.ds(base, E_PER_W)], idx_v)

    def step(g, carry):
        hs = []
        for b in range(NBUF):
            off = (g * NBUF + b) * GCHUNK
            hs.append(pltpu.async_copy(
                bm_hbm.at[idx_v.at[pl.ds(off, GCHUNK)]], bufs[b], sem))
        for h in hs:
            h.wait()
        ws = []
        for b in range(NBUF):
            off = (g * NBUF + b) * GCHUNK
            ws.append(pltpu.async_copy(
                bufs[b], out_hbm.at[pl.ds(base + off, GCHUNK)], sem))
        for w in ws:
            w.wait()
        return carry

    lax.fori_loop(0, N_CHUNKS // NBUF, step, 0)


def kernel(pos, features, W1, b1, W2, b2):
    B, N, _ = pos.shape
    p = pos.reshape(BN, 3)
    f = features.reshape(BN, F)
    P = jnp.pad(p, ((0, 0), (0, 5)))
    PT = -2.0 * P.T
    W1a = W1[:, :F]
    W1b = W1[:, F:]
    Wd = (W1a - W1b).T
    Wb = W1b.T
    W2T = W2.T
    b1r = b1.reshape(1, F)
    b2r = b2.reshape(1, F)

    knn, A, Bm = pl.pallas_call(
        _dist_topk_body,
        grid=(N_TILES,),
        in_specs=[
            pl.BlockSpec((ROWS, 8), lambda i: (i, 0)),
            pl.BlockSpec((8, BN), lambda i: (0, 0)),
            pl.BlockSpec((ROWS, F), lambda i: (i, 0)),
            pl.BlockSpec((F, F), lambda i: (0, 0)),
            pl.BlockSpec((F, F), lambda i: (0, 0)),
            pl.BlockSpec((1, F), lambda i: (0, 0)),
        ],
        out_specs=[
            pl.BlockSpec((ROWS, K), lambda i: (i, 0)),
            pl.BlockSpec((ROWS, F), lambda i: (i, 0)),
            pl.BlockSpec((ROWS, F), lambda i: (i, 0)),
        ],
        out_shape=[
            jax.ShapeDtypeStruct((BN, K), jnp.int32),
            jax.ShapeDtypeStruct((BN, F), jnp.float32),
            jax.ShapeDtypeStruct((BN, F), jnp.float32),
        ],
    )(P, PT, f, Wd, Wb, b1r)

    idx_flat = knn.reshape(EDGES)

    gather = functools.partial(
        pl.kernel,
        out_type=jax.ShapeDtypeStruct((EDGES, F), jnp.float32),
        mesh=plsc.VectorSubcoreMesh(core_axis_name="c", subcore_axis_name="s"),
        scratch_types=[
            pltpu.VMEM((E_PER_W,), jnp.int32),
            pltpu.VMEM((GCHUNK, F), jnp.float32),
            pltpu.VMEM((GCHUNK, F), jnp.float32),
            pltpu.VMEM((GCHUNK, F), jnp.float32),
            pltpu.VMEM((GCHUNK, F), jnp.float32),
            pltpu.SemaphoreType.DMA,
        ],
    )(_sc_gather)
    G = gather(Bm, idx_flat)

    out = pl.pallas_call(
        _edge_mlp_body,
        grid=(BN // CROWS,),
        in_specs=[
            pl.BlockSpec((CROWS * K, F), lambda i: (i, 0)),
            pl.BlockSpec((CROWS, F), lambda i: (i, 0)),
            pl.BlockSpec((F, F), lambda i: (0, 0)),
            pl.BlockSpec((1, F), lambda i: (0, 0)),
        ],
        out_specs=pl.BlockSpec((CROWS, F), lambda i: (i, 0)),
        out_shape=jax.ShapeDtypeStruct((BN, F), jnp.float32),
    )(G, A, W2T, b2r)

    return out.reshape(B, N, F)
